# baseline (speedup 1.0000x reference)
import functools

import jax
import jax.numpy as jnp
from jax import lax
from jax.experimental import pallas as pl
from jax.experimental.pallas import tpu as pltpu
from jax.experimental.pallas import tpu_sc as plsc

_N = 10000
_D = 128
_E = 320000
_NC = 2
_NS = 16
_CH = 128
_Q = 4
_NBLK = -(-_E // (_CH * _NS * _Q))
_NPW = _NBLK * _Q
_EPAD = _NPW * _CH * _NS
_RT = 640
_NA = _RT * _NS
_HPAD = 8


def _sc_two_sided_accumulate(h_pad, eidx, zero_blk):
    mesh = plsc.VectorSubcoreMesh(core_axis_name="c", subcore_axis_name="s")

    @functools.partial(
        pl.kernel,
        out_type=jax.ShapeDtypeStruct((_NC, _NA, _D), jnp.float32),
        mesh=mesh,
        scratch_types=[
            pltpu.VMEM((_Q, _CH), jnp.int32),
            pltpu.VMEM((_Q, _CH), jnp.int32),
            pltpu.VMEM((_CH, _D), jnp.float32),
            pltpu.VMEM((_CH, _D), jnp.float32),
            pltpu.VMEM_SHARED((_NA, _D), jnp.float32),
            *[pltpu.SemaphoreType.DMA for _ in range(4)],
        ],
    )
    def k(h_hbm, e_hbm, z_hbm, out_hbm, sg, sd, r0, r1, acc, *sems):
        rows = [r0, r1]
        gsem = sems[0:2]
        ssem = sems[2:4]
        c = lax.axis_index("c")
        s = lax.axis_index("s")

        pltpu.sync_copy(z_hbm, acc.at[pl.ds(s * _RT, _RT)])
        plsc.subcore_barrier()

        def body(blk, carry):
            pltpu.sync_copy(e_hbm.at[c, s, blk], sg)
            pltpu.sync_copy(e_hbm.at[1, s, blk], sd)
            cps = {0: pltpu.async_copy(h_hbm.at[sg.at[0]], rows[0], gsem[0])}
            scs = {}
            for u in range(_Q):
                if u + 1 < _Q:
                    if u >= 1:
                        scs[u - 1].wait()
                    cps[u + 1] = pltpu.async_copy(
                        h_hbm.at[sg.at[u + 1]], rows[(u + 1) % 2],
                        gsem[(u + 1) % 2])
                cps[u].wait()
                scs[u] = pltpu.async_copy(
                    rows[u % 2], acc.at[sd.at[u]], ssem[u % 2], add=True)
            scs[_Q - 2].wait()
            scs[_Q - 1].wait()
            return carry

        lax.fori_loop(0, _NBLK, body, 0)
        plsc.subcore_barrier()

        pltpu.sync_copy(acc.at[pl.ds(s * _RT, _RT)],
                        out_hbm.at[c, pl.ds(s * _RT, _RT)])

    return k(h_pad, eidx, zero_blk)


_BN = 400


def _tc_finalize(partials):
    def body(p_ref, o_ref):
        agg = p_ref[0] - p_ref[1]
        ss = jnp.sum(agg * agg, axis=1, keepdims=True)
        o_ref[...] = agg / (jnp.sqrt(ss) + 1e-7)

    return pl.pallas_call(
        body,
        grid=(_N // _BN,),
        in_specs=[pl.BlockSpec((_NC, _BN, _D), lambda i: (0, i, 0))],
        out_specs=pl.BlockSpec((_BN, _D), lambda i: (i, 0)),
        out_shape=jax.ShapeDtypeStruct((_N, _D), jnp.float32),
    )(partials)


def kernel(h, edge_index, r):
    eidx = jnp.concatenate(
        [edge_index.astype(jnp.int32),
         jnp.full((2, _EPAD - _E), _N, jnp.int32)], axis=1)
    eidx = eidx.reshape(2, _NS, _NBLK, _Q, _CH)
    h_pad = jnp.concatenate(
        [h, jnp.zeros((_HPAD, _D), jnp.float32)], axis=0)
    zero_blk = jnp.zeros((_RT, _D), jnp.float32)
    partials = _sc_two_sided_accumulate(h_pad, eidx, zero_blk)
    return _tc_finalize(partials)
